# Initial kernel scaffold; baseline (speedup 1.0000x reference)
import functools

import jax
import jax.numpy as jnp
from jax import lax
from jax.experimental import pallas as pl
from jax.experimental.pallas import tpu as pltpu
from jax.experimental.pallas import tpu_sc as plsc

N_NODES = 10000
N_EDGES = 320000
D_FEAT = 128
MISC_W = 16

NUM_CORES = 2
NUM_SUBCORES = 16
NW = NUM_CORES * NUM_SUBCORES

EDGES_PER_TILE = N_EDGES // NW
CHUNK = 128
N_FULL_CHUNKS = EDGES_PER_TILE // CHUNK
TAIL = EDGES_PER_TILE - N_FULL_CHUNKS * CHUNK
ROWS_PER_TILE = N_NODES // NUM_SUBCORES


def _sc_segment_sums(x, col, row, misc):
  mesh = plsc.VectorSubcoreMesh(
      core_axis_name="c", subcore_axis_name="s",
      num_cores=NUM_CORES, num_subcores=NUM_SUBCORES)

  @functools.partial(
      pl.kernel,
      out_type=[
          jax.ShapeDtypeStruct((NUM_CORES * N_NODES, D_FEAT), jnp.float32),
          jax.ShapeDtypeStruct((NUM_CORES * N_NODES, MISC_W), jnp.float32),
      ],
      mesh=mesh,
      scratch_types=[
          pltpu.VMEM((CHUNK,), jnp.int32),
          pltpu.VMEM((CHUNK,), jnp.int32),
          pltpu.VMEM((CHUNK, MISC_W), jnp.float32),
          pltpu.VMEM((CHUNK, D_FEAT), jnp.float32),
          pltpu.VMEM((TAIL,), jnp.int32),
          pltpu.VMEM((TAIL,), jnp.int32),
          pltpu.VMEM((TAIL, MISC_W), jnp.float32),
          pltpu.VMEM((TAIL, D_FEAT), jnp.float32),
          pltpu.VMEM_SHARED((N_NODES, D_FEAT), jnp.float32),
          pltpu.VMEM_SHARED((N_NODES, MISC_W), jnp.float32),
          pltpu.SemaphoreType.DMA,
      ],
  )
  def seg_kernel(x_hbm, col_hbm, row_hbm, misc_hbm, feat_out, misc_out,
                 colv, rowv, miscv, rowsv, colt, rowt, misct, rowst,
                 acc_feat, acc_misc, sem):
    cid = lax.axis_index("c")
    sid = lax.axis_index("s")
    wid = sid * NUM_CORES + cid
    e0 = wid * EDGES_PER_TILE
    zeros16 = jnp.zeros((16,), jnp.float32)

    @pl.loop(0, CHUNK)
    def _zero(i):
      for j in range(D_FEAT // 16):
        rowsv[i, pl.ds(j * 16, 16)] = zeros16
      miscv[i, :] = zeros16

    r0 = sid * ROWS_PER_TILE
    n_zero_full = ROWS_PER_TILE // CHUNK
    for k in range(n_zero_full):
      pltpu.sync_copy(rowsv, acc_feat.at[pl.ds(r0 + k * CHUNK, CHUNK)])
      pltpu.sync_copy(miscv, acc_misc.at[pl.ds(r0 + k * CHUNK, CHUNK)])
    rem = ROWS_PER_TILE - n_zero_full * CHUNK
    if rem:
      pltpu.sync_copy(rowsv.at[pl.ds(0, rem)],
                      acc_feat.at[pl.ds(r0 + n_zero_full * CHUNK, rem)])
      pltpu.sync_copy(miscv.at[pl.ds(0, rem)],
                      acc_misc.at[pl.ds(r0 + n_zero_full * CHUNK, rem)])
    plsc.subcore_barrier()

    @pl.loop(0, N_FULL_CHUNKS)
    def _chunk(i):
      base = e0 + i * CHUNK
      pltpu.sync_copy(col_hbm.at[pl.ds(base, CHUNK)], colv)
      pltpu.sync_copy(row_hbm.at[pl.ds(base, CHUNK)], rowv)
      pltpu.sync_copy(misc_hbm.at[pl.ds(base, CHUNK)], miscv)
      pltpu.async_copy(x_hbm.at[colv], rowsv, sem).wait()
      pltpu.sync_copy(rowsv, acc_feat.at[rowv], add=True)
      pltpu.sync_copy(miscv, acc_misc.at[rowv], add=True)

    if TAIL:
      tbase = e0 + N_FULL_CHUNKS * CHUNK
      pltpu.sync_copy(col_hbm.at[pl.ds(tbase, TAIL)], colt)
      pltpu.sync_copy(row_hbm.at[pl.ds(tbase, TAIL)], rowt)
      pltpu.sync_copy(misc_hbm.at[pl.ds(tbase, TAIL)], misct)
      pltpu.async_copy(x_hbm.at[colt], rowst, sem).wait()
      pltpu.sync_copy(rowst, acc_feat.at[rowt], add=True)
      pltpu.sync_copy(misct, acc_misc.at[rowt], add=True)

    plsc.subcore_barrier()

    o0 = cid * N_NODES + r0
    pltpu.sync_copy(acc_feat.at[pl.ds(r0, ROWS_PER_TILE)],
                    feat_out.at[pl.ds(o0, ROWS_PER_TILE)])
    pltpu.sync_copy(acc_misc.at[pl.ds(r0, ROWS_PER_TILE)],
                    misc_out.at[pl.ds(o0, ROWS_PER_TILE)])

  return seg_kernel(x, col, row, misc)


BLK = 400


def _tc_body(f_ref, m_ref, x_ref, w1_ref, w2_ref, w3_ref, b_ref, o_ref):
  f = f_ref[0] + f_ref[1]
  m = m_ref[0] + m_ref[1]
  inv = 1.0 / jnp.maximum(m[:, 4:5], 1.0)
  acc = jnp.dot(f * inv, w1_ref[...], preferred_element_type=jnp.float32,
                precision=lax.Precision.HIGHEST)
  acc = acc + jnp.dot(m * inv, w2_ref[...], preferred_element_type=jnp.float32,
                      precision=lax.Precision.HIGHEST)
  acc = acc + jnp.dot(x_ref[...], w3_ref[...], preferred_element_type=jnp.float32,
                      precision=lax.Precision.HIGHEST)
  o_ref[...] = acc + b_ref[...]


def _tc_linear(feat_part, misc_part, x, weight, bias):
  w1 = weight[:D_FEAT]
  w2 = jnp.zeros((MISC_W, weight.shape[1]), jnp.float32).at[:4].set(
      weight[D_FEAT:D_FEAT + 4])
  w3 = weight[D_FEAT + 4:]
  b = bias.reshape(1, -1)
  grid = N_NODES // BLK
  return pl.pallas_call(
      _tc_body,
      grid=(grid,),
      in_specs=[
          pl.BlockSpec((NUM_CORES, BLK, D_FEAT), lambda i: (0, i, 0)),
          pl.BlockSpec((NUM_CORES, BLK, MISC_W), lambda i: (0, i, 0)),
          pl.BlockSpec((BLK, D_FEAT), lambda i: (i, 0)),
          pl.BlockSpec((D_FEAT, D_FEAT), lambda i: (0, 0)),
          pl.BlockSpec((MISC_W, D_FEAT), lambda i: (0, 0)),
          pl.BlockSpec((D_FEAT, D_FEAT), lambda i: (0, 0)),
          pl.BlockSpec((1, D_FEAT), lambda i: (0, 0)),
      ],
      out_specs=pl.BlockSpec((BLK, D_FEAT), lambda i: (i, 0)),
      out_shape=jax.ShapeDtypeStruct((N_NODES, D_FEAT), jnp.float32),
  )(feat_part, misc_part, x, w1, w2, w3, b)


def kernel(x, edge_index, edge_label, weight, bias):
  row = edge_index[0].astype(jnp.int32)
  col = edge_index[1].astype(jnp.int32)
  misc = jnp.concatenate(
      [edge_label,
       jnp.ones((N_EDGES, 1), jnp.float32),
       jnp.zeros((N_EDGES, MISC_W - 5), jnp.float32)], axis=1)
  feat_part, misc_part = _sc_segment_sums(x, col, row, misc)
  feat_part = feat_part.reshape(NUM_CORES, N_NODES, D_FEAT)
  misc_part = misc_part.reshape(NUM_CORES, N_NODES, MISC_W)
  return _tc_linear(feat_part, misc_part, x, weight, bias)


# speedup vs baseline: 1.0148x; 1.0148x over previous
import jax
import jax.numpy as jnp
from jax import lax
from jax.experimental import pallas as pl
from jax.experimental.pallas import tpu as pltpu

N_NODES = 10000
N_EDGES = 320000
D_FEAT = 128
MISC_W = 8

ECHUNK = 128
N_CHUNKS = N_EDGES // ECHUNK


def _seg_body(idx_ref, ml_ref, x_ref, feat_ref, misc_ref):
  pid = pl.program_id(0)

  @pl.when(pid == 0)
  def _init():
    feat_ref[...] = jnp.zeros_like(feat_ref)
    misc_ref[...] = jnp.zeros_like(misc_ref)

  def body(e, _):
    r = idx_ref[0, 0, e]
    c = idx_ref[0, 1, e]
    feat_ref[pl.ds(r, 1), :] = feat_ref[pl.ds(r, 1), :] + x_ref[pl.ds(c, 1), :]
    misc_ref[pl.ds(r, 1), :] = misc_ref[pl.ds(r, 1), :] + ml_ref[0, pl.ds(e, 1), :]
    return _

  lax.fori_loop(0, ECHUNK, body, None)


def _segment_sums(x, idx3, misc3):
  return pl.pallas_call(
      _seg_body,
      grid=(N_CHUNKS,),
      in_specs=[
          pl.BlockSpec((1, 2, ECHUNK), lambda i: (i, 0, 0),
                       memory_space=pltpu.SMEM),
          pl.BlockSpec((1, ECHUNK, MISC_W), lambda i: (i, 0, 0)),
          pl.BlockSpec((N_NODES, D_FEAT), lambda i: (0, 0)),
      ],
      out_specs=[
          pl.BlockSpec((N_NODES, D_FEAT), lambda i: (0, 0)),
          pl.BlockSpec((N_NODES, MISC_W), lambda i: (0, 0)),
      ],
      out_shape=[
          jax.ShapeDtypeStruct((N_NODES, D_FEAT), jnp.float32),
          jax.ShapeDtypeStruct((N_NODES, MISC_W), jnp.float32),
      ],
  )(idx3, misc3, x)


BLK = 400


def _tc_body(f_ref, m_ref, x_ref, w1_ref, w2_ref, w3_ref, b_ref, o_ref):
  f = f_ref[...]
  m = m_ref[...]
  inv = 1.0 / jnp.maximum(m[:, 4:5], 1.0)
  acc = jnp.dot(f * inv, w1_ref[...], preferred_element_type=jnp.float32,
                precision=lax.Precision.HIGHEST)
  acc = acc + jnp.dot(m * inv, w2_ref[...], preferred_element_type=jnp.float32,
                      precision=lax.Precision.HIGHEST)
  acc = acc + jnp.dot(x_ref[...], w3_ref[...], preferred_element_type=jnp.float32,
                      precision=lax.Precision.HIGHEST)
  o_ref[...] = acc + b_ref[...]


def _tc_linear(feat_acc, misc_acc, x, weight, bias):
  w1 = weight[:D_FEAT]
  w2 = jnp.zeros((MISC_W, weight.shape[1]), jnp.float32).at[:4].set(
      weight[D_FEAT:D_FEAT + 4])
  w3 = weight[D_FEAT + 4:]
  b = bias.reshape(1, -1)
  return pl.pallas_call(
      _tc_body,
      grid=(N_NODES // BLK,),
      in_specs=[
          pl.BlockSpec((BLK, D_FEAT), lambda i: (i, 0)),
          pl.BlockSpec((BLK, MISC_W), lambda i: (i, 0)),
          pl.BlockSpec((BLK, D_FEAT), lambda i: (i, 0)),
          pl.BlockSpec((D_FEAT, D_FEAT), lambda i: (0, 0)),
          pl.BlockSpec((MISC_W, D_FEAT), lambda i: (0, 0)),
          pl.BlockSpec((D_FEAT, D_FEAT), lambda i: (0, 0)),
          pl.BlockSpec((1, D_FEAT), lambda i: (0, 0)),
      ],
      out_specs=pl.BlockSpec((BLK, D_FEAT), lambda i: (i, 0)),
      out_shape=jax.ShapeDtypeStruct((N_NODES, D_FEAT), jnp.float32),
  )(feat_acc, misc_acc, x, w1, w2, w3, b)


def kernel(x, edge_index, edge_label, weight, bias):
  idx3 = edge_index.astype(jnp.int32).reshape(2, N_CHUNKS, ECHUNK).transpose(1, 0, 2)
  misc3 = jnp.concatenate(
      [edge_label,
       jnp.ones((N_EDGES, 1), jnp.float32),
       jnp.zeros((N_EDGES, MISC_W - 5), jnp.float32)],
      axis=1).reshape(N_CHUNKS, ECHUNK, MISC_W)
  feat_acc, misc_acc = _segment_sums(x, idx3, misc3)
  return _tc_linear(feat_acc, misc_acc, x, weight, bias)


# speedup vs baseline: 1.4063x; 1.3858x over previous
import jax
import jax.numpy as jnp
from jax import lax
from jax.experimental import pallas as pl
from jax.experimental.pallas import tpu as pltpu

N_NODES = 10000
N_EDGES = 320000
D_FEAT = 128
MISC_W = 8

ECHUNK = 128
N_CHUNKS = N_EDGES // ECHUNK


def _seg_body(idx_ref, ml_ref, x_ref, feat_ref, misc_ref):
  pid = pl.program_id(0)

  @pl.when(pid == 0)
  def _init():
    feat_ref[...] = jnp.zeros_like(feat_ref)
    misc_ref[...] = jnp.zeros_like(misc_ref)

  def body(e, _):
    r = idx_ref[0, 0, e]
    c = idx_ref[0, 1, e]
    feat_ref[pl.ds(r, 1), :] = feat_ref[pl.ds(r, 1), :] + x_ref[pl.ds(c, 1), :]
    misc_ref[pl.ds(r, 1), :] = misc_ref[pl.ds(r, 1), :] + ml_ref[0, pl.ds(e, 1), :]
    return _

  lax.fori_loop(0, ECHUNK, body, None, unroll=8)


def _segment_sums(x, idx3, misc3):
  return pl.pallas_call(
      _seg_body,
      grid=(N_CHUNKS,),
      in_specs=[
          pl.BlockSpec((1, 2, ECHUNK), lambda i: (i, 0, 0),
                       memory_space=pltpu.SMEM),
          pl.BlockSpec((1, ECHUNK, MISC_W), lambda i: (i, 0, 0)),
          pl.BlockSpec((N_NODES, D_FEAT), lambda i: (0, 0)),
      ],
      out_specs=[
          pl.BlockSpec((N_NODES, D_FEAT), lambda i: (0, 0)),
          pl.BlockSpec((N_NODES, MISC_W), lambda i: (0, 0)),
      ],
      out_shape=[
          jax.ShapeDtypeStruct((N_NODES, D_FEAT), jnp.float32),
          jax.ShapeDtypeStruct((N_NODES, MISC_W), jnp.float32),
      ],
  )(idx3, misc3, x)


BLK = 400


def _tc_body(f_ref, m_ref, x_ref, w1_ref, w2_ref, w3_ref, b_ref, o_ref):
  f = f_ref[...]
  m = m_ref[...]
  inv = 1.0 / jnp.maximum(m[:, 4:5], 1.0)
  acc = jnp.dot(f * inv, w1_ref[...], preferred_element_type=jnp.float32,
                precision=lax.Precision.HIGHEST)
  acc = acc + jnp.dot(m * inv, w2_ref[...], preferred_element_type=jnp.float32,
                      precision=lax.Precision.HIGHEST)
  acc = acc + jnp.dot(x_ref[...], w3_ref[...], preferred_element_type=jnp.float32,
                      precision=lax.Precision.HIGHEST)
  o_ref[...] = acc + b_ref[...]


def _tc_linear(feat_acc, misc_acc, x, weight, bias):
  w1 = weight[:D_FEAT]
  w2 = jnp.zeros((MISC_W, weight.shape[1]), jnp.float32).at[:4].set(
      weight[D_FEAT:D_FEAT + 4])
  w3 = weight[D_FEAT + 4:]
  b = bias.reshape(1, -1)
  return pl.pallas_call(
      _tc_body,
      grid=(N_NODES // BLK,),
      in_specs=[
          pl.BlockSpec((BLK, D_FEAT), lambda i: (i, 0)),
          pl.BlockSpec((BLK, MISC_W), lambda i: (i, 0)),
          pl.BlockSpec((BLK, D_FEAT), lambda i: (i, 0)),
          pl.BlockSpec((D_FEAT, D_FEAT), lambda i: (0, 0)),
          pl.BlockSpec((MISC_W, D_FEAT), lambda i: (0, 0)),
          pl.BlockSpec((D_FEAT, D_FEAT), lambda i: (0, 0)),
          pl.BlockSpec((1, D_FEAT), lambda i: (0, 0)),
      ],
      out_specs=pl.BlockSpec((BLK, D_FEAT), lambda i: (i, 0)),
      out_shape=jax.ShapeDtypeStruct((N_NODES, D_FEAT), jnp.float32),
  )(feat_acc, misc_acc, x, w1, w2, w3, b)


def kernel(x, edge_index, edge_label, weight, bias):
  idx3 = edge_index.astype(jnp.int32).reshape(2, N_CHUNKS, ECHUNK).transpose(1, 0, 2)
  misc3 = jnp.concatenate(
      [edge_label,
       jnp.ones((N_EDGES, 1), jnp.float32),
       jnp.zeros((N_EDGES, MISC_W - 5), jnp.float32)],
      axis=1).reshape(N_CHUNKS, ECHUNK, MISC_W)
  feat_acc, misc_acc = _segment_sums(x, idx3, misc3)
  return _tc_linear(feat_acc, misc_acc, x, weight, bias)


# speedup vs baseline: 1.4119x; 1.0040x over previous
import jax
import jax.numpy as jnp
from jax import lax
from jax.experimental import pallas as pl
from jax.experimental.pallas import tpu as pltpu

N_NODES = 10000
N_EDGES = 320000
D_FEAT = 128
MISC_W = 8

ECHUNK = 128
N_CHUNKS = N_EDGES // ECHUNK


def _seg_body(idx_ref, ml_ref, x_ref, feat_ref, misc_ref):
  pid = pl.program_id(0)

  @pl.when(pid == 0)
  def _init():
    feat_ref[...] = jnp.zeros_like(feat_ref)
    misc_ref[...] = jnp.zeros_like(misc_ref)

  def body(e, _):
    r = idx_ref[0, 0, e]
    c = idx_ref[0, 1, e]
    feat_ref[pl.ds(r, 1), :] = feat_ref[pl.ds(r, 1), :] + x_ref[pl.ds(c, 1), :]
    misc_ref[pl.ds(r, 1), :] = misc_ref[pl.ds(r, 1), :] + ml_ref[0, pl.ds(e, 1), :]
    return _

  lax.fori_loop(0, ECHUNK, body, None, unroll=32)


def _segment_sums(x, idx3, misc3):
  return pl.pallas_call(
      _seg_body,
      grid=(N_CHUNKS,),
      in_specs=[
          pl.BlockSpec((1, 2, ECHUNK), lambda i: (i, 0, 0),
                       memory_space=pltpu.SMEM),
          pl.BlockSpec((1, ECHUNK, MISC_W), lambda i: (i, 0, 0)),
          pl.BlockSpec((N_NODES, D_FEAT), lambda i: (0, 0)),
      ],
      out_specs=[
          pl.BlockSpec((N_NODES, D_FEAT), lambda i: (0, 0)),
          pl.BlockSpec((N_NODES, MISC_W), lambda i: (0, 0)),
      ],
      out_shape=[
          jax.ShapeDtypeStruct((N_NODES, D_FEAT), jnp.float32),
          jax.ShapeDtypeStruct((N_NODES, MISC_W), jnp.float32),
      ],
  )(idx3, misc3, x)


BLK = 400


def _tc_body(f_ref, m_ref, x_ref, w1_ref, w2_ref, w3_ref, b_ref, o_ref):
  f = f_ref[...]
  m = m_ref[...]
  inv = 1.0 / jnp.maximum(m[:, 4:5], 1.0)
  acc = jnp.dot(f * inv, w1_ref[...], preferred_element_type=jnp.float32,
                precision=lax.Precision.HIGHEST)
  acc = acc + jnp.dot(m * inv, w2_ref[...], preferred_element_type=jnp.float32,
                      precision=lax.Precision.HIGHEST)
  acc = acc + jnp.dot(x_ref[...], w3_ref[...], preferred_element_type=jnp.float32,
                      precision=lax.Precision.HIGHEST)
  o_ref[...] = acc + b_ref[...]


def _tc_linear(feat_acc, misc_acc, x, weight, bias):
  w1 = weight[:D_FEAT]
  w2 = jnp.zeros((MISC_W, weight.shape[1]), jnp.float32).at[:4].set(
      weight[D_FEAT:D_FEAT + 4])
  w3 = weight[D_FEAT + 4:]
  b = bias.reshape(1, -1)
  return pl.pallas_call(
      _tc_body,
      grid=(N_NODES // BLK,),
      in_specs=[
          pl.BlockSpec((BLK, D_FEAT), lambda i: (i, 0)),
          pl.BlockSpec((BLK, MISC_W), lambda i: (i, 0)),
          pl.BlockSpec((BLK, D_FEAT), lambda i: (i, 0)),
          pl.BlockSpec((D_FEAT, D_FEAT), lambda i: (0, 0)),
          pl.BlockSpec((MISC_W, D_FEAT), lambda i: (0, 0)),
          pl.BlockSpec((D_FEAT, D_FEAT), lambda i: (0, 0)),
          pl.BlockSpec((1, D_FEAT), lambda i: (0, 0)),
      ],
      out_specs=pl.BlockSpec((BLK, D_FEAT), lambda i: (i, 0)),
      out_shape=jax.ShapeDtypeStruct((N_NODES, D_FEAT), jnp.float32),
  )(feat_acc, misc_acc, x, w1, w2, w3, b)


def kernel(x, edge_index, edge_label, weight, bias):
  idx3 = edge_index.astype(jnp.int32).reshape(2, N_CHUNKS, ECHUNK).transpose(1, 0, 2)
  misc3 = jnp.concatenate(
      [edge_label,
       jnp.ones((N_EDGES, 1), jnp.float32),
       jnp.zeros((N_EDGES, MISC_W - 5), jnp.float32)],
      axis=1).reshape(N_CHUNKS, ECHUNK, MISC_W)
  feat_acc, misc_acc = _segment_sums(x, idx3, misc3)
  return _tc_linear(feat_acc, misc_acc, x, weight, bias)
